# baseline (speedup 1.0000x reference)
import jax
import jax.numpy as jnp
from jax import lax
from jax.experimental import pallas as pl
from jax.experimental.pallas import tpu as pltpu
from jax.experimental.pallas import tpu_sc as plsc

_N = 10000
_E = 320000
_D = 128
_NC, _NS = 2, 16
_NW = _NC * _NS
_EPW = _E // _NW
_C = 80
_NCH = _EPW // _C
_NP = 5
_CP = _NCH // _NP
_CR = 80
_NPAD = 10112
_RPS = _NPAD // _NS
_NB = 3


def _sc_body(x_hbm, edge_hbm, zeros_hbm, zeros1_hbm, out_hbm, cnt_hbm,
             src_v, dst_v, rows_v, hist_v, acc_sh,
             gsem, ssem, isem):
    c = lax.axis_index("c")
    s = lax.axis_index("s")
    wid = s * _NC + c
    ebase = wid * _EPW

    pltpu.sync_copy(zeros_hbm, acc_sh.at[pl.ds(s * _RPS, _RPS)])
    pltpu.sync_copy(zeros1_hbm, hist_v)
    ones16 = jnp.full((16,), 1.0, dtype=jnp.float32)
    plsc.subcore_barrier()

    def _stage(p, pb):
        pbase = ebase + p * (_CP * _C)
        pltpu.async_copy(edge_hbm.at[0, pl.ds(pbase, _CP * _C)],
                         src_v.at[pb], isem)
        for k in range(_CP):
            pltpu.async_copy(edge_hbm.at[1, pl.ds(pbase + k * _C, _C)],
                             dst_v.at[pb, k], isem)

    def _stage_wait(pb):
        pltpu.make_async_copy(edge_hbm.at[0, pl.ds(ebase, _CP * _C)],
                              src_v.at[pb], isem).wait()
        for k in range(_CP):
            pltpu.make_async_copy(edge_hbm.at[1, pl.ds(ebase, _C)],
                                  dst_v.at[pb, k], isem).wait()

    _stage(0, 0)
    _stage_wait(0)
    pltpu.async_copy(x_hbm.at[src_v.at[0, pl.ds(0, _C)]], rows_v.at[0],
                     gsem)
    pltpu.async_copy(x_hbm.at[src_v.at[0, pl.ds(_C, _C)]], rows_v.at[1],
                     gsem)

    @pl.loop(0, _NCH)
    def _(j):
        q = lax.div(j, _CP)
        k = lax.rem(j, _CP)
        pb = lax.rem(q, 2)
        b = lax.rem(j, _NB)

        @pl.when((k == 1) & (j + _CP < _NCH))
        def _():
            _stage(q + 1, 1 - pb)

        for i in range(_C // 16):
            d16 = dst_v[pb, k, pl.ds(i * 16, 16)]
            plsc.addupdate_scatter(hist_v, [d16], ones16)

        pltpu.make_async_copy(x_hbm.at[src_v.at[pb, pl.ds(k * _C, _C)]],
                              rows_v.at[b], gsem).wait()
        pltpu.async_copy(rows_v.at[b], acc_sh.at[dst_v.at[pb, k]], ssem,
                         add=True)

        @pl.when((k == _CP - 2) & (j + 2 < _NCH))
        def _():
            _stage_wait(1 - pb)

        @pl.when(j + 2 < _NCH)
        def _():
            bn = lax.rem(j + 2, _NB)

            @pl.when(j >= 1)
            def _():
                pltpu.make_async_copy(rows_v.at[bn],
                                      acc_sh.at[dst_v.at[0, 0]],
                                      ssem).wait()
            j2 = j + 2
            q2 = lax.div(j2, _CP)
            k2 = lax.rem(j2, _CP)
            pltpu.async_copy(
                x_hbm.at[src_v.at[lax.rem(q2, 2), pl.ds(k2 * _C, _C)]],
                rows_v.at[bn], gsem)

    for b in range(_NB):
        pltpu.make_async_copy(rows_v.at[b], acc_sh.at[dst_v.at[0, 0]],
                              ssem).wait()

    pltpu.sync_copy(hist_v, cnt_hbm.at[c, s])
    plsc.subcore_barrier()
    pltpu.sync_copy(acc_sh.at[pl.ds(s * _RPS, _RPS)],
                    out_hbm.at[c, pl.ds(s * _RPS, _RPS)])


_sc_agg_cache = []


def _sc_agg(*arrs):
    if not _sc_agg_cache:
        _sc_agg_cache.append(pl.kernel(
            _sc_body,
            out_type=(jax.ShapeDtypeStruct((_NC, _NPAD, _D), jnp.float32),
                      jax.ShapeDtypeStruct((_NC, _NS, _CR * _D),
                                           jnp.float32)),
            mesh=plsc.VectorSubcoreMesh(core_axis_name="c",
                                        subcore_axis_name="s",
                                        num_cores=_NC, num_subcores=_NS),
            scratch_types=[
                pltpu.VMEM((2, _CP * _C), jnp.int32),
                pltpu.VMEM((2, _CP, _C), jnp.int32),
                pltpu.VMEM((_NB, _C, _D), jnp.float32),
                pltpu.VMEM((_CR * _D,), jnp.float32),
                pltpu.VMEM_SHARED((_NPAD, _D), jnp.float32),
                pltpu.SemaphoreType.DMA,
                pltpu.SemaphoreType.DMA,
                pltpu.SemaphoreType.DMA,
            ],
            compiler_params=pltpu.CompilerParams(use_tc_tiling_on_sc=False,
                                                 needs_layout_passes=False),
        ))
    return _sc_agg_cache[0](*arrs)


def _tc_r_body(x_ref, wr_ref, bl_ref, o_ref):
    o_ref[...] = lax.dot_general(x_ref[...], wr_ref[...],
                                 (((1,), (1,)), ((), ())),
                                 preferred_element_type=jnp.float32
                                 ) + bl_ref[...]


def _tc_body(p_ref, h_ref, x_ref, wl_ref, xr_ref, g_ref, b_ref, o_ref):
    sums = p_ref[0, :_N] + p_ref[1, :_N]
    cflat = jnp.sum(h_ref[...], axis=(0, 1))
    cnt = cflat[:_N].reshape(_N, 1)
    mean = sums / jnp.maximum(cnt, 1.0)
    x = x_ref[...]
    out = (lax.dot_general(mean, wl_ref[...], (((1,), (1,)), ((), ())),
                           preferred_element_type=jnp.float32)
           + xr_ref[...])
    mu = jnp.mean(out, axis=0, keepdims=True)
    var = jnp.mean((out - mu) * (out - mu), axis=0, keepdims=True)
    y = (out - mu) * lax.rsqrt(var + 1e-5) * g_ref[...] + b_ref[...]
    o_ref[...] = jnp.maximum(y, 0.0) + x


def kernel(x, edge_index, W_l, b_l, W_r, gamma, beta):
    zeros = jnp.zeros((_RPS, _D), jnp.float32)
    zeros1 = jnp.zeros((_CR * _D,), jnp.float32)
    p, h = _sc_agg(x, edge_index, zeros, zeros1)
    xr = pl.pallas_call(
        _tc_r_body,
        out_shape=jax.ShapeDtypeStruct((_N, _D), jnp.float32),
    )(x, W_r, b_l.reshape(1, _D))
    return pl.pallas_call(
        _tc_body,
        out_shape=jax.ShapeDtypeStruct((_N, _D), jnp.float32),
    )(p, h, x, W_l, xr, gamma.reshape(1, _D), beta.reshape(1, _D))


# speedup vs baseline: 1.0294x; 1.0294x over previous
import jax
import jax.numpy as jnp
from jax import lax
from jax.experimental import pallas as pl
from jax.experimental.pallas import tpu as pltpu
from jax.experimental.pallas import tpu_sc as plsc

_N = 10000
_E = 320000
_D = 128
_NC, _NS = 2, 16
_NW = _NC * _NS
_EPW = _E // _NW
_C = 80
_NCH = _EPW // _C
_NP = 5
_CP = _NCH // _NP
_CR = 80
_NPAD = 10112
_RPS = _NPAD // _NS
_NB = 3


def _sc_body(x_hbm, edge_hbm, zeros_hbm, zeros1_hbm, out_hbm, cnt_hbm,
             src_v, dst_v, rows_v, hist_v, acc_sh,
             gsem, ssem, isem):
    c = lax.axis_index("c")
    s = lax.axis_index("s")
    wid = s * _NC + c
    ebase = wid * _EPW

    ones16 = jnp.full((16,), 1.0, dtype=jnp.float32)

    def _stage(p, pb):
        pbase = ebase + p * (_CP * _C)
        pltpu.async_copy(edge_hbm.at[0, pl.ds(pbase, _CP * _C)],
                         src_v.at[pb], isem)
        for k in range(_CP):
            pltpu.async_copy(edge_hbm.at[1, pl.ds(pbase + k * _C, _C)],
                             dst_v.at[pb, k], isem)

    def _stage_wait(pb):
        pltpu.make_async_copy(edge_hbm.at[0, pl.ds(ebase, _CP * _C)],
                              src_v.at[pb], isem).wait()
        for k in range(_CP):
            pltpu.make_async_copy(edge_hbm.at[1, pl.ds(ebase, _C)],
                                  dst_v.at[pb, k], isem).wait()

    pltpu.async_copy(zeros_hbm, acc_sh.at[pl.ds(s * _RPS, _RPS)], ssem)
    _stage(0, 0)
    pltpu.sync_copy(zeros1_hbm, hist_v)
    pltpu.make_async_copy(zeros_hbm, acc_sh.at[pl.ds(s * _RPS, _RPS)],
                          ssem).wait()
    _stage_wait(0)
    pltpu.async_copy(x_hbm.at[src_v.at[0, pl.ds(0, _C)]], rows_v.at[0],
                     gsem)
    pltpu.async_copy(x_hbm.at[src_v.at[0, pl.ds(_C, _C)]], rows_v.at[1],
                     gsem)
    plsc.subcore_barrier()

    @pl.loop(0, _NCH)
    def _(j):
        q = lax.div(j, _CP)
        k = lax.rem(j, _CP)
        pb = lax.rem(q, 2)
        b = lax.rem(j, _NB)

        @pl.when((k == 1) & (j + _CP < _NCH))
        def _():
            _stage(q + 1, 1 - pb)

        for i in range(_C // 16):
            d16 = dst_v[pb, k, pl.ds(i * 16, 16)]
            plsc.addupdate_scatter(hist_v, [d16], ones16)

        pltpu.make_async_copy(x_hbm.at[src_v.at[pb, pl.ds(k * _C, _C)]],
                              rows_v.at[b], gsem).wait()
        pltpu.async_copy(rows_v.at[b], acc_sh.at[dst_v.at[pb, k]], ssem,
                         add=True)

        @pl.when((k == _CP - 2) & (j + 2 < _NCH))
        def _():
            _stage_wait(1 - pb)

        @pl.when(j + 2 < _NCH)
        def _():
            bn = lax.rem(j + 2, _NB)

            @pl.when(j >= 1)
            def _():
                pltpu.make_async_copy(rows_v.at[bn],
                                      acc_sh.at[dst_v.at[0, 0]],
                                      ssem).wait()
            j2 = j + 2
            q2 = lax.div(j2, _CP)
            k2 = lax.rem(j2, _CP)
            pltpu.async_copy(
                x_hbm.at[src_v.at[lax.rem(q2, 2), pl.ds(k2 * _C, _C)]],
                rows_v.at[bn], gsem)

    pltpu.async_copy(hist_v, cnt_hbm.at[c, s], gsem)
    for b in range(_NB):
        pltpu.make_async_copy(rows_v.at[b], acc_sh.at[dst_v.at[0, 0]],
                              ssem).wait()
    pltpu.make_async_copy(hist_v, cnt_hbm.at[c, s], gsem).wait()
    plsc.subcore_barrier()
    pltpu.sync_copy(acc_sh.at[pl.ds(s * _RPS, _RPS)],
                    out_hbm.at[c, pl.ds(s * _RPS, _RPS)])


_sc_agg_cache = []


def _sc_agg(*arrs):
    if not _sc_agg_cache:
        _sc_agg_cache.append(pl.kernel(
            _sc_body,
            out_type=(jax.ShapeDtypeStruct((_NC, _NPAD, _D), jnp.float32),
                      jax.ShapeDtypeStruct((_NC, _NS, _CR * _D),
                                           jnp.float32)),
            mesh=plsc.VectorSubcoreMesh(core_axis_name="c",
                                        subcore_axis_name="s",
                                        num_cores=_NC, num_subcores=_NS),
            scratch_types=[
                pltpu.VMEM((2, _CP * _C), jnp.int32),
                pltpu.VMEM((2, _CP, _C), jnp.int32),
                pltpu.VMEM((_NB, _C, _D), jnp.float32),
                pltpu.VMEM((_CR * _D,), jnp.float32),
                pltpu.VMEM_SHARED((_NPAD, _D), jnp.float32),
                pltpu.SemaphoreType.DMA,
                pltpu.SemaphoreType.DMA,
                pltpu.SemaphoreType.DMA,
            ],
            compiler_params=pltpu.CompilerParams(use_tc_tiling_on_sc=False,
                                                 needs_layout_passes=False),
        ))
    return _sc_agg_cache[0](*arrs)


def _tc_body(p_ref, h_ref, x_ref, wl_ref, bl_ref, wr_ref, g_ref, b_ref,
             o_ref):
    sums = p_ref[0, :_N] + p_ref[1, :_N]
    cflat = jnp.sum(h_ref[...], axis=(0, 1))
    cnt = cflat[:_N].reshape(_N, 1)
    mean = sums / jnp.maximum(cnt, 1.0)
    x = x_ref[...]
    out = (lax.dot_general(mean, wl_ref[...], (((1,), (1,)), ((), ())),
                           preferred_element_type=jnp.float32)
           + bl_ref[...]
           + lax.dot_general(x, wr_ref[...], (((1,), (1,)), ((), ())),
                             preferred_element_type=jnp.float32))
    mu = jnp.mean(out, axis=0, keepdims=True)
    var = jnp.mean((out - mu) * (out - mu), axis=0, keepdims=True)
    y = (out - mu) * lax.rsqrt(var + 1e-5) * g_ref[...] + b_ref[...]
    o_ref[...] = jnp.maximum(y, 0.0) + x


def kernel(x, edge_index, W_l, b_l, W_r, gamma, beta):
    zeros = jnp.zeros((_RPS, _D), jnp.float32)
    zeros1 = jnp.zeros((_CR * _D,), jnp.float32)
    p, h = _sc_agg(x, edge_index, zeros, zeros1)
    return pl.pallas_call(
        _tc_body,
        out_shape=jax.ShapeDtypeStruct((_N, _D), jnp.float32),
    )(p, h, x, W_l, b_l.reshape(1, _D), W_r,
      gamma.reshape(1, _D), beta.reshape(1, _D))
